# baseline (speedup 1.0000x reference)
import functools

import jax
import jax.numpy as jnp
from jax import lax
from jax.experimental import pallas as pl
from jax.experimental.pallas import tpu as pltpu
from jax.experimental.pallas import tpu_sc as plsc

_N = 1024
_D = 512
_BLK = 256
_BIG = 2**30


def _mine_body(x_blk_ref, x_all_ref, t_col_ref, t_row_ref, pos_ref, neg_ref):
    x_blk = x_blk_ref[...]
    x_all = x_all_ref[...]
    t_col = t_col_ref[...]
    t_row = t_row_ref[...]
    dot = lax.dot_general(x_blk, x_all, (((1,), (1,)), ((), ())),
                          preferred_element_type=jnp.float32)
    sq_col = jnp.sum(x_blk * x_blk, axis=1, keepdims=True)
    sq_row = jnp.sum(x_all * x_all, axis=1)[None, :]
    d2 = (sq_col + sq_row) - 2.0 * dot
    dist = jnp.sqrt(jnp.clip(d2, 1e-12, None))
    mask = t_col == t_row
    ids = lax.broadcasted_iota(jnp.int32, (_BLK, _N), 1)
    pos_d = jnp.where(mask, dist, -jnp.inf)
    pmax = jnp.max(pos_d, axis=1, keepdims=True)
    pos_ref[...] = jnp.min(jnp.where(pos_d == pmax, ids, _BIG),
                           axis=1, keepdims=True)
    neg_d = jnp.where(mask, jnp.inf, dist)
    nmin = jnp.min(neg_d, axis=1, keepdims=True)
    neg_ref[...] = jnp.min(jnp.where(neg_d == nmin, ids, _BIG),
                           axis=1, keepdims=True)


def _mine(x, t_col, t_row, interpret=False):
    return pl.pallas_call(
        _mine_body,
        grid=(_N // _BLK,),
        in_specs=[
            pl.BlockSpec((_BLK, _D), lambda i: (i, 0)),
            pl.BlockSpec((_N, _D), lambda i: (0, 0)),
            pl.BlockSpec((_BLK, 1), lambda i: (i, 0)),
            pl.BlockSpec((1, _N), lambda i: (0, 0)),
        ],
        out_specs=[
            pl.BlockSpec((_BLK, 1), lambda i: (i, 0)),
            pl.BlockSpec((_BLK, 1), lambda i: (i, 0)),
        ],
        out_shape=[
            jax.ShapeDtypeStruct((_N, 1), jnp.int32),
            jax.ShapeDtypeStruct((_N, 1), jnp.int32),
        ],
        interpret=interpret,
    )(x, x, t_col, t_row)


@functools.cache
def _sc_gather_fn():
    info = plsc.get_sparse_core_info()
    nc, ns = info.num_cores, info.num_subcores
    nw = nc * ns
    b = 2 * _N
    bpw = b // nw
    mesh = plsc.VectorSubcoreMesh(core_axis_name="c", subcore_axis_name="s")

    @functools.partial(
        pl.kernel,
        mesh=mesh,
        out_type=jax.ShapeDtypeStruct((b, _D), jnp.float32),
        scratch_types=[
            pltpu.VMEM((bpw,), jnp.int32),
            pltpu.VMEM((bpw, _D), jnp.float32),
            pltpu.SemaphoreType.DMA,
        ],
    )
    def gather(table_hbm, idx_hbm, out_hbm, idx_v, rows_v, sem):
        wid = lax.axis_index("s") * nc + lax.axis_index("c")
        base = wid * bpw
        pltpu.sync_copy(idx_hbm.at[pl.ds(base, bpw)], idx_v)
        pltpu.async_copy(table_hbm.at[idx_v], rows_v, sem).wait()
        pltpu.sync_copy(rows_v, out_hbm.at[pl.ds(base, bpw)])

    return gather


def kernel(inputs, targets):
    t_col = targets.reshape(_N, 1)
    t_row = targets.reshape(1, _N)
    pos, neg = _mine(inputs, t_col, t_row)
    idx = jnp.concatenate([pos[:, 0], neg[:, 0]], axis=0)
    pair2 = _sc_gather_fn()(inputs, idx)
    pair1 = jnp.concatenate([inputs, inputs], axis=0)
    y = jnp.concatenate([jnp.ones((_N,), inputs.dtype),
                         jnp.zeros((_N,), inputs.dtype)], axis=0)
    return (pair1, pair2, y)


# speedup vs baseline: 2.7903x; 2.7903x over previous
import functools

import jax
import jax.numpy as jnp
from jax import lax
from jax.experimental import pallas as pl
from jax.experimental.pallas import tpu as pltpu
from jax.experimental.pallas import tpu_sc as plsc

_N = 1024
_D = 512
_BLK = 256
_BIG = 2**30


def _body(x_blk_ref, x_all_ref, t_col_ref, t_row_ref, pair1_ref, pair2_ref):
    x_blk = x_blk_ref[...]
    x_all = x_all_ref[...]
    t_col = t_col_ref[...]
    t_row = t_row_ref[...]
    dot = lax.dot_general(x_blk, x_all, (((1,), (1,)), ((), ())),
                          preferred_element_type=jnp.float32)
    sq_col = jnp.sum(x_blk * x_blk, axis=1, keepdims=True)
    sq_row = jnp.sum(x_all * x_all, axis=1)[None, :]
    d2 = (sq_col + sq_row) - 2.0 * dot
    dist = jnp.sqrt(jnp.clip(d2, 1e-12, None))
    mask = t_col == t_row
    ids = lax.broadcasted_iota(jnp.int32, (_BLK, _N), 1)
    pos_d = jnp.where(mask, dist, -jnp.inf)
    pmax = jnp.max(pos_d, axis=1, keepdims=True)
    pos_idx = jnp.min(jnp.where(pos_d == pmax, ids, _BIG), axis=1,
                      keepdims=True)
    neg_d = jnp.where(mask, jnp.inf, dist)
    nmin = jnp.min(neg_d, axis=1, keepdims=True)
    neg_idx = jnp.min(jnp.where(neg_d == nmin, ids, _BIG), axis=1,
                      keepdims=True)
    onehot_p = (ids == pos_idx).astype(jnp.float32)
    onehot_n = (ids == neg_idx).astype(jnp.float32)
    pair2_ref[0] = lax.dot_general(onehot_p, x_all, (((1,), (0,)), ((), ())),
                                   preferred_element_type=jnp.float32)
    pair2_ref[1] = lax.dot_general(onehot_n, x_all, (((1,), (0,)), ((), ())),
                                   preferred_element_type=jnp.float32)
    pair1_ref[0] = x_blk
    pair1_ref[1] = x_blk


def _fused(x, t_col, t_row, interpret=False):
    return pl.pallas_call(
        _body,
        grid=(_N // _BLK,),
        in_specs=[
            pl.BlockSpec((_BLK, _D), lambda i: (i, 0)),
            pl.BlockSpec((_N, _D), lambda i: (0, 0)),
            pl.BlockSpec((_BLK, 1), lambda i: (i, 0)),
            pl.BlockSpec((1, _N), lambda i: (0, 0)),
        ],
        out_specs=[
            pl.BlockSpec((2, _BLK, _D), lambda i: (0, i, 0)),
            pl.BlockSpec((2, _BLK, _D), lambda i: (0, i, 0)),
        ],
        out_shape=[
            jax.ShapeDtypeStruct((2, _N, _D), jnp.float32),
            jax.ShapeDtypeStruct((2, _N, _D), jnp.float32),
        ],
        interpret=interpret,
    )(x, x, t_col, t_row)


def kernel(inputs, targets):
    t_col = targets.reshape(_N, 1)
    t_row = targets.reshape(1, _N)
    pair1, pair2 = _fused(inputs, t_col, t_row)
    y = jnp.concatenate([jnp.ones((_N,), inputs.dtype),
                         jnp.zeros((_N,), inputs.dtype)], axis=0)
    return (pair1.reshape(2 * _N, _D), pair2.reshape(2 * _N, _D), y)
